# baseline (speedup 1.0000x reference)
import functools

import jax
import jax.numpy as jnp
from jax import lax
from jax.experimental import pallas as pl
from jax.experimental.pallas import tpu as pltpu
from jax.experimental.pallas import tpu_sc as plsc

LD, LU, N, D_IN, D_OUT = 16384, 65536, 2, 256, 128


_MLP_BLK = 2048


def _mlp_body(x_ref, w1_ref, b1_ref, w2_ref, b2_ref, o0_ref, o1_ref):
    for n, o_ref in ((0, o0_ref), (1, o1_ref)):
        x = x_ref[:, n, :]
        h = jnp.dot(x, w1_ref[...], preferred_element_type=jnp.float32)
        h = jnp.maximum(h + b1_ref[...], 0.0)
        o_ref[...] = jnp.dot(h, w2_ref[...], preferred_element_type=jnp.float32) + b2_ref[...]


def _mlp(down, W1, b1, W2, b2):
    return pl.pallas_call(
        _mlp_body,
        grid=(LD // _MLP_BLK,),
        in_specs=[
            pl.BlockSpec((_MLP_BLK, N, D_IN), lambda i: (i, 0, 0)),
            pl.BlockSpec((D_IN, D_OUT), lambda i: (0, 0)),
            pl.BlockSpec((1, D_OUT), lambda i: (0, 0)),
            pl.BlockSpec((D_OUT, D_OUT), lambda i: (0, 0)),
            pl.BlockSpec((1, D_OUT), lambda i: (0, 0)),
        ],
        out_specs=[
            pl.BlockSpec((_MLP_BLK, D_OUT), lambda i: (i, 0)),
            pl.BlockSpec((_MLP_BLK, D_OUT), lambda i: (i, 0)),
        ],
        out_shape=[
            jax.ShapeDtypeStruct((LD, D_OUT), jnp.float32),
            jax.ShapeDtypeStruct((LD, D_OUT), jnp.float32),
        ],
    )(down, W1, b1.reshape(1, D_OUT), W2, b2.reshape(1, D_OUT))


_NC, _NS = 2, 16
_NW = _NC * _NS
_CF = 64
_STEPS = LU // (_NW * _CF)
_F_PER_W = LU // _NW
_YCHUNK = 128
_LANES = 16
_GBUF = 3
_OBUF = 4
_AHEAD = 3


def _sc_gather_add(h0, h1, Y, up3):
    mesh = plsc.VectorSubcoreMesh(
        core_axis_name="c", subcore_axis_name="s",
        num_cores=_NC, num_subcores=_NS)

    @functools.partial(
        pl.kernel,
        out_type=jax.ShapeDtypeStruct((LU, N, D_OUT), jnp.float32),
        mesh=mesh,
        scratch_types=[
            pltpu.VMEM((2 * _F_PER_W // _YCHUNK, _YCHUNK), jnp.int32),
            pltpu.VMEM((_GBUF, _CF, D_OUT), jnp.float32),
            pltpu.VMEM((_GBUF, _CF, D_OUT), jnp.float32),
            pltpu.VMEM((_OBUF, _CF, N, D_OUT), jnp.float32),
            [pltpu.SemaphoreType.DMA] * _GBUF,
            [pltpu.SemaphoreType.DMA] * _OBUF,
            [pltpu.SemaphoreType.DMA] * _OBUF,
        ],
    )
    def k(h0_hbm, h1_hbm, y_hbm, up_hbm, out_hbm,
          iy, g0, g1, ob, gsems, lsems, ssems):
        wid = lax.axis_index("s") * _NC + lax.axis_index("c")
        yrow0 = wid * (2 * _F_PER_W // _YCHUNK)
        fbase = wid * _F_PER_W
        pltpu.sync_copy(y_hbm.at[pl.ds(yrow0, 2 * _F_PER_W // _YCHUNK)], iy)

        def issue_loads(i):
            s2, s3 = i % _GBUF, i % _OBUF
            f0 = fbase + _CF * i
            lc, hh = i // 2, i % 2
            return (
                pltpu.async_copy(up_hbm.at[pl.ds(f0, _CF)], ob.at[s3], lsems[s3]),
                pltpu.async_copy(h0_hbm.at[iy.at[2 * lc, pl.ds(_CF * hh, _CF)]],
                                 g0.at[s2], gsems[s2]),
                pltpu.async_copy(h1_hbm.at[iy.at[2 * lc + 1, pl.ds(_CF * hh, _CF)]],
                                 g1.at[s2], gsems[s2]),
            )

        def add_into(o_ref, g0_ref, g1_ref):
            def body(r, c):
                for t in range(D_OUT // _LANES):
                    sl = pl.ds(_LANES * t, _LANES)
                    plsc.addupdate(o_ref.at[r, 0, sl], g0_ref[r, sl])
                    plsc.addupdate(o_ref.at[r, 1, sl], g1_ref[r, sl])
                return c
            lax.fori_loop(0, _CF, body, 0)

        loads = {i: issue_loads(i) for i in range(_AHEAD)}
        stores = {}
        for i in range(_STEPS):
            s2, s3 = i % _GBUF, i % _OBUF
            for d in loads.pop(i):
                d.wait()
            add_into(ob.at[s3], g0.at[s2], g1.at[s2])
            f0 = fbase + _CF * i
            stores[i] = (
                pltpu.async_copy(ob.at[s3], out_hbm.at[pl.ds(f0, _CF)], ssems[s3]),
            )
            ni = i + _AHEAD
            if ni < _STEPS:
                if ni - _OBUF in stores:
                    for d in stores.pop(ni - _OBUF):
                        d.wait()
                loads[ni] = issue_loads(ni)
        for i in sorted(stores):
            for d in stores[i]:
                d.wait()

    return k(h0, h1, Y, up3)


def kernel(down_features, up_features, indices, W1, b1, W2, b2):
    h0, h1 = _mlp(down_features, W1, b1, W2, b2)
    Y = (indices.astype(jnp.int32)
         .reshape(LU // _YCHUNK, _YCHUNK, N)
         .swapaxes(1, 2)
         .reshape(2 * LU // _YCHUNK, _YCHUNK))
    return _sc_gather_add(h0, h1, Y, up_features)


# speedup vs baseline: 1.4040x; 1.4040x over previous
import functools

import jax
import jax.numpy as jnp
from jax import lax
from jax.experimental import pallas as pl
from jax.experimental.pallas import tpu as pltpu
from jax.experimental.pallas import tpu_sc as plsc

LD, LU, N, D_IN, D_OUT = 16384, 65536, 2, 256, 128


_MLP_BLK = 2048


def _mlp_body(x_ref, w1_ref, b1_ref, w2_ref, b2_ref, o0_ref, o1_ref):
    for n, o_ref in ((0, o0_ref), (1, o1_ref)):
        x = x_ref[:, n, :]
        h = jnp.dot(x, w1_ref[...], preferred_element_type=jnp.float32)
        h = jnp.maximum(h + b1_ref[...], 0.0)
        o_ref[...] = jnp.dot(h, w2_ref[...], preferred_element_type=jnp.float32) + b2_ref[...]


def _mlp(down, W1, b1, W2, b2):
    return pl.pallas_call(
        _mlp_body,
        grid=(LD // _MLP_BLK,),
        in_specs=[
            pl.BlockSpec((_MLP_BLK, N, D_IN), lambda i: (i, 0, 0)),
            pl.BlockSpec((D_IN, D_OUT), lambda i: (0, 0)),
            pl.BlockSpec((1, D_OUT), lambda i: (0, 0)),
            pl.BlockSpec((D_OUT, D_OUT), lambda i: (0, 0)),
            pl.BlockSpec((1, D_OUT), lambda i: (0, 0)),
        ],
        out_specs=[
            pl.BlockSpec((_MLP_BLK, D_OUT), lambda i: (i, 0)),
            pl.BlockSpec((_MLP_BLK, D_OUT), lambda i: (i, 0)),
        ],
        out_shape=[
            jax.ShapeDtypeStruct((LD, D_OUT), jnp.float32),
            jax.ShapeDtypeStruct((LD, D_OUT), jnp.float32),
        ],
    )(down, W1, b1.reshape(1, D_OUT), W2, b2.reshape(1, D_OUT))


_NC, _NS = 2, 16
_NW = _NC * _NS
_CF = 64
_STEPS = LU // (_NW * _CF)
_F_PER_W = LU // _NW
_YCHUNK = 128
_LANES = 16
_GBUF = 3
_OBUF = 4
_AHEAD = 3


def _sc_gather_add(h0, h1, Y, up3):
    mesh = plsc.VectorSubcoreMesh(
        core_axis_name="c", subcore_axis_name="s",
        num_cores=_NC, num_subcores=_NS)

    @functools.partial(
        pl.kernel,
        out_type=jax.ShapeDtypeStruct((LU, N, D_OUT), jnp.float32),
        mesh=mesh,
        scratch_types=[
            pltpu.VMEM((2 * _F_PER_W // _YCHUNK, _YCHUNK), jnp.int32),
            pltpu.VMEM((_GBUF, _CF, D_OUT), jnp.float32),
            pltpu.VMEM((_GBUF, _CF, D_OUT), jnp.float32),
            pltpu.VMEM((_OBUF, _CF, D_OUT), jnp.float32),
            pltpu.VMEM((_OBUF, _CF, D_OUT), jnp.float32),
            [pltpu.SemaphoreType.DMA] * _GBUF,
            [pltpu.SemaphoreType.DMA] * _OBUF,
            [pltpu.SemaphoreType.DMA] * _OBUF,
        ],
    )
    def k(h0_hbm, h1_hbm, y_hbm, up_hbm, out_hbm,
          iy, g0, g1, o0, o1, gsems, lsems, ssems):
        wid = lax.axis_index("s") * _NC + lax.axis_index("c")
        yrow0 = wid * (2 * _F_PER_W // _YCHUNK)
        fbase = wid * _F_PER_W
        pltpu.sync_copy(y_hbm.at[pl.ds(yrow0, 2 * _F_PER_W // _YCHUNK)], iy)

        def issue_loads(i):
            s2, s3 = i % _GBUF, i % _OBUF
            f0 = fbase + _CF * i
            lc, hh = i // 2, i % 2
            return (
                pltpu.async_copy(up_hbm.at[pl.ds(f0, _CF), 0], o0.at[s3], lsems[s3]),
                pltpu.async_copy(up_hbm.at[pl.ds(f0, _CF), 1], o1.at[s3], lsems[s3]),
                pltpu.async_copy(h0_hbm.at[iy.at[2 * lc, pl.ds(_CF * hh, _CF)]],
                                 g0.at[s2], gsems[s2]),
                pltpu.async_copy(h1_hbm.at[iy.at[2 * lc + 1, pl.ds(_CF * hh, _CF)]],
                                 g1.at[s2], gsems[s2]),
            )

        def add_into(o_ref, g_ref):
            def body(r2, c):
                for dr in range(2):
                    r = 2 * r2 + dr
                    for t in range(D_OUT // _LANES):
                        sl = pl.ds(_LANES * t, _LANES)
                        plsc.addupdate(o_ref.at[r, sl], g_ref[r, sl])
                return c
            lax.fori_loop(0, _CF // 2, body, 0)

        loads = {i: issue_loads(i) for i in range(_AHEAD)}
        stores = {}
        for i in range(_STEPS):
            s2, s3 = i % _GBUF, i % _OBUF
            for d in loads.pop(i):
                d.wait()
            add_into(o0.at[s3], g0.at[s2])
            add_into(o1.at[s3], g1.at[s2])
            f0 = fbase + _CF * i
            stores[i] = (
                pltpu.async_copy(o0.at[s3], out_hbm.at[pl.ds(f0, _CF), 0], ssems[s3]),
                pltpu.async_copy(o1.at[s3], out_hbm.at[pl.ds(f0, _CF), 1], ssems[s3]),
            )
            ni = i + _AHEAD
            if ni < _STEPS:
                if ni - _OBUF in stores:
                    for d in stores.pop(ni - _OBUF):
                        d.wait()
                loads[ni] = issue_loads(ni)
        for i in sorted(stores):
            for d in stores[i]:
                d.wait()

    return k(h0, h1, Y, up3)


def kernel(down_features, up_features, indices, W1, b1, W2, b2):
    h0, h1 = _mlp(down_features, W1, b1, W2, b2)
    Y = (indices.astype(jnp.int32)
         .reshape(LU // _YCHUNK, _YCHUNK, N)
         .swapaxes(1, 2)
         .reshape(2 * LU // _YCHUNK, _YCHUNK))
    return _sc_gather_add(h0, h1, Y, up_features)


# speedup vs baseline: 1.4347x; 1.0218x over previous
import functools

import jax
import jax.numpy as jnp
from jax import lax
from jax.experimental import pallas as pl
from jax.experimental.pallas import tpu as pltpu
from jax.experimental.pallas import tpu_sc as plsc

LD, LU, N, D_IN, D_OUT = 16384, 65536, 2, 256, 128


_MLP_BLK = 2048


def _mlp_body(x_ref, w1_ref, b1_ref, w2_ref, b2_ref, o0_ref, o1_ref):
    for n, o_ref in ((0, o0_ref), (1, o1_ref)):
        x = x_ref[:, n, :]
        h = jnp.dot(x, w1_ref[...], preferred_element_type=jnp.float32)
        h = jnp.maximum(h + b1_ref[...], 0.0)
        o_ref[...] = jnp.dot(h, w2_ref[...], preferred_element_type=jnp.float32) + b2_ref[...]


def _mlp(down, W1, b1, W2, b2):
    return pl.pallas_call(
        _mlp_body,
        grid=(LD // _MLP_BLK,),
        in_specs=[
            pl.BlockSpec((_MLP_BLK, N, D_IN), lambda i: (i, 0, 0)),
            pl.BlockSpec((D_IN, D_OUT), lambda i: (0, 0)),
            pl.BlockSpec((1, D_OUT), lambda i: (0, 0)),
            pl.BlockSpec((D_OUT, D_OUT), lambda i: (0, 0)),
            pl.BlockSpec((1, D_OUT), lambda i: (0, 0)),
        ],
        out_specs=[
            pl.BlockSpec((_MLP_BLK, D_OUT), lambda i: (i, 0)),
            pl.BlockSpec((_MLP_BLK, D_OUT), lambda i: (i, 0)),
        ],
        out_shape=[
            jax.ShapeDtypeStruct((LD, D_OUT), jnp.float32),
            jax.ShapeDtypeStruct((LD, D_OUT), jnp.float32),
        ],
    )(down, W1, b1.reshape(1, D_OUT), W2, b2.reshape(1, D_OUT))


_NC, _NS = 2, 16
_NW = _NC * _NS
_CF = 64
_STEPS = LU // (_NW * _CF)
_F_PER_W = LU // _NW
_YCHUNK = 128
_LANES = 16
_GBUF = 3
_OBUF = 4
_AHEAD = 3


def _sc_gather_add(h0, h1, Y, up3):
    mesh = plsc.VectorSubcoreMesh(
        core_axis_name="c", subcore_axis_name="s",
        num_cores=_NC, num_subcores=_NS)

    @functools.partial(
        pl.kernel,
        out_type=jax.ShapeDtypeStruct((LU, N, D_OUT), jnp.float32),
        mesh=mesh,
        scratch_types=[
            pltpu.VMEM((2 * _F_PER_W // _YCHUNK, _YCHUNK), jnp.int32),
            pltpu.VMEM((_GBUF, _CF, D_OUT), jnp.float32),
            pltpu.VMEM((_GBUF, _CF, D_OUT), jnp.float32),
            pltpu.VMEM((_OBUF, _CF, D_OUT), jnp.float32),
            pltpu.VMEM((_OBUF, _CF, D_OUT), jnp.float32),
            [pltpu.SemaphoreType.DMA] * _GBUF,
            [pltpu.SemaphoreType.DMA] * _OBUF,
            [pltpu.SemaphoreType.DMA] * _OBUF,
        ],
    )
    def k(h0_hbm, h1_hbm, y_hbm, up_hbm, out_hbm,
          iy, g0, g1, o0, o1, gsems, lsems, ssems):
        wid = lax.axis_index("s") * _NC + lax.axis_index("c")
        yrow0 = wid * (2 * _F_PER_W // _YCHUNK)
        fbase = wid * _F_PER_W
        pltpu.sync_copy(y_hbm.at[pl.ds(yrow0, 2 * _F_PER_W // _YCHUNK)], iy)

        def issue_loads(i):
            s2, s3 = i % _GBUF, i % _OBUF
            f0 = fbase + _CF * i
            lc, hh = i // 2, i % 2
            return (
                pltpu.async_copy(up_hbm.at[pl.ds(f0, _CF), 0], o0.at[s3], lsems[s3]),
                pltpu.async_copy(up_hbm.at[pl.ds(f0, _CF), 1], o1.at[s3], lsems[s3]),
                pltpu.async_copy(h0_hbm.at[iy.at[2 * lc, pl.ds(_CF * hh, _CF)]],
                                 g0.at[s2], gsems[s2]),
                pltpu.async_copy(h1_hbm.at[iy.at[2 * lc + 1, pl.ds(_CF * hh, _CF)]],
                                 g1.at[s2], gsems[s2]),
            )

        def add_into(o_ref, g_ref):
            def body(r, c):
                for t in range(D_OUT // _LANES):
                    sl = pl.ds(_LANES * t, _LANES)
                    plsc.addupdate(o_ref.at[r, sl], g_ref[r, sl])
                return c
            lax.fori_loop(0, _CF, body, 0)

        loads = {i: issue_loads(i) for i in range(_AHEAD)}
        stores = {}
        for i in range(_STEPS):
            s2, s3 = i % _GBUF, i % _OBUF
            for d in loads.pop(i):
                d.wait()
            add_into(o0.at[s3], g0.at[s2])
            add_into(o1.at[s3], g1.at[s2])
            f0 = fbase + _CF * i
            stores[i] = (
                pltpu.async_copy(o0.at[s3], out_hbm.at[pl.ds(f0, _CF), 0], ssems[s3]),
                pltpu.async_copy(o1.at[s3], out_hbm.at[pl.ds(f0, _CF), 1], ssems[s3]),
            )
            ni = i + _AHEAD
            if ni < _STEPS:
                if ni - _OBUF in stores:
                    for d in stores.pop(ni - _OBUF):
                        d.wait()
                loads[ni] = issue_loads(ni)
        for i in sorted(stores):
            for d in stores[i]:
                d.wait()

    return k(h0, h1, Y, up3)


def kernel(down_features, up_features, indices, W1, b1, W2, b2):
    h0, h1 = _mlp(down_features, W1, b1, W2, b2)
    Y = (indices.astype(jnp.int32)
         .reshape(LU // _YCHUNK, _YCHUNK, N)
         .swapaxes(1, 2)
         .reshape(2 * LU // _YCHUNK, _YCHUNK))
    return _sc_gather_add(h0, h1, Y, up_features)
